# baseline (speedup 1.0000x reference)
import functools

import jax
import jax.numpy as jnp
from jax import lax
from jax.experimental import pallas as pl
from jax.experimental.pallas import tpu as pltpu
from jax.experimental.pallas import tpu_sc as plsc

SIGMA = 9.0
NC = 2
NS = 16
NW = NC * NS
L = 16
BLK = 128
U = BLK // L

CA = 6400
NB = CA // BLK


def _make_sc_partials(n_anchors: int, interpret: bool = False):
    assert n_anchors % CA == 0
    n_chunks = n_anchors // CA

    mesh = plsc.VectorSubcoreMesh(
        core_axis_name="c", subcore_axis_name="s", num_cores=NC,
        num_subcores=NS)

    @functools.partial(
        pl.kernel,
        mesh=mesh,
        out_type=jax.ShapeDtypeStruct((NW, 2, L), jnp.float32),
        scratch_types=[
            pltpu.VMEM((2, NB, 2, BLK), jnp.float32),
            pltpu.VMEM((2, CA), jnp.float32),
            pltpu.VMEM((2, CA), jnp.float32),
            pltpu.VMEM((2, CA), jnp.float32),
            pltpu.VMEM((2, L), jnp.float32),
            pltpu.SemaphoreType.DMA((2,)),
        ],
        interpret=interpret,
        compiler_params=pltpu.CompilerParams(
            needs_layout_passes=False, use_tc_tiling_on_sc=True),
    )
    def sc_partials(inp_hbm, tgt_hbm, out_hbm, in_v, cls_v, r0_v, r1_v,
                    out_v, sem):
        wid = lax.axis_index("s") * NC + lax.axis_index("c")
        start_c = (wid * n_chunks) // NW
        end_c = ((wid + 1) * n_chunks) // NW

        inv_sigma = jnp.float32(1.0 / SIGMA)
        half_sigma = jnp.float32(0.5 * SIGMA)
        one = jnp.float32(1.0)
        zero = jnp.zeros((L,), jnp.float32)

        def chunk_copies(c):
            d = lax.rem(c, 2)
            return (
                pltpu.make_async_copy(
                    inp_hbm.at[pl.ds(c * NB, NB), :, :], in_v.at[d],
                    sem.at[d]),
                pltpu.make_async_copy(
                    tgt_hbm.at[0, 0, pl.ds(c * CA, CA)], cls_v.at[d],
                    sem.at[d]),
                pltpu.make_async_copy(
                    tgt_hbm.at[1, 0, pl.ds(c * CA, CA)], r0_v.at[d],
                    sem.at[d]),
                pltpu.make_async_copy(
                    tgt_hbm.at[2, 0, pl.ds(c * CA, CA)], r1_v.at[d],
                    sem.at[d]),
            )

        def start_chunk(c):
            for cp in chunk_copies(c):
                cp.start()

        def wait_chunk(c):
            for cp in chunk_copies(c):
                cp.wait()

        start_chunk(start_c)

        def chunk_body(c, carry):
            @pl.when(c + 1 < end_c)
            def _():
                start_chunk(c + 1)

            wait_chunk(c)
            d = lax.rem(c, 2)

            def blk_body(g, carry2):
                accs = list(carry2[:U])
                cnts = list(carry2[U:])
                for u in range(U):
                    cls = cls_v[d, pl.ds(g * BLK + u * L, L)]
                    r0 = r0_v[d, pl.ds(g * BLK + u * L, L)]
                    r1 = r1_v[d, pl.ds(g * BLK + u * L, L)]
                    p0 = in_v[d, g, 0, pl.ds(u * L, L)]
                    p1 = in_v[d, g, 1, pl.ds(u * L, L)]
                    d0 = jnp.abs(r0 - p0)
                    d1 = jnp.abs(r1 - p1)
                    t0 = jnp.minimum(d0, inv_sigma)
                    t1 = jnp.minimum(d1, inv_sigma)
                    w = ((d0 - t0) + (d1 - t1)
                         + half_sigma * (t0 * t0 + t1 * t1))
                    m = cls == one
                    accs[u] = jnp.where(m, accs[u] + w, accs[u])
                    cnts[u] = jnp.where(m, cnts[u] + one, cnts[u])
                return tuple(accs) + tuple(cnts)

            return lax.fori_loop(0, NB, blk_body, carry)

        init = (zero,) * (2 * U)
        fin = lax.fori_loop(start_c, end_c, chunk_body, init)
        acc = fin[0]
        cnt = fin[U]
        for u in range(1, U):
            acc = acc + fin[u]
            cnt = cnt + fin[U + u]
        out_v[0, :] = acc
        out_v[1, :] = cnt
        pltpu.sync_copy(out_v, out_hbm.at[wid])

    return sc_partials


def kernel(input, target):
    n = input.shape[1]
    tgt_pl = jnp.transpose(target, (2, 0, 1))
    inp_pl = input.reshape(n // BLK, BLK, 2).transpose(0, 2, 1)
    partials = _make_sc_partials(n)(inp_pl, tgt_pl)
    s = jnp.sum(partials[:, 0, :])
    c = jnp.sum(partials[:, 1, :])
    return jnp.where(c > 0, s / jnp.maximum(c, 1.0), jnp.float32(0.0))


# speedup vs baseline: 1.7300x; 1.7300x over previous
import functools

import jax
import jax.numpy as jnp
from jax.experimental import pallas as pl
from jax.experimental.pallas import tpu as pltpu

SIGMA = 9.0
BLK = 128

TCRB = 1250


def _make_tc_partials(n_anchors: int):
    nrows = n_anchors // BLK
    assert nrows % TCRB == 0
    grid = nrows // TCRB
    ca = TCRB * BLK

    def body(in_ref, cls_ref, r0_ref, r1_ref, out_ref):
        i = pl.program_id(0)

        @pl.when(i == 0)
        def _():
            out_ref[0] = jnp.float32(0.0)
            out_ref[1] = jnp.float32(0.0)

        p0 = in_ref[:, 0, :]
        p1 = in_ref[:, 1, :]
        cls = cls_ref[0, 0, :].reshape(TCRB, BLK)
        r0 = r0_ref[0, 0, :].reshape(TCRB, BLK)
        r1 = r1_ref[0, 0, :].reshape(TCRB, BLK)
        d0 = jnp.abs(r0 - p0)
        d1 = jnp.abs(r1 - p1)
        c = jnp.float32(1.0 / SIGMA)
        t0 = jnp.minimum(d0, c)
        t1 = jnp.minimum(d1, c)
        w = (d0 - t0) + (d1 - t1) + jnp.float32(0.5 * SIGMA) * (
            t0 * t0 + t1 * t1)
        mf = jnp.where(cls == 1.0, jnp.float32(1.0), jnp.float32(0.0))
        out_ref[0] += jnp.sum(w * mf)
        out_ref[1] += jnp.sum(mf)

    return pl.pallas_call(
        body,
        grid=(grid,),
        in_specs=[
            pl.BlockSpec((TCRB, 2, BLK), lambda i: (i, 0, 0)),
            pl.BlockSpec((1, 1, ca), lambda i: (0, 0, i)),
            pl.BlockSpec((1, 1, ca), lambda i: (1, 0, i)),
            pl.BlockSpec((1, 1, ca), lambda i: (2, 0, i)),
        ],
        out_specs=pl.BlockSpec(memory_space=pltpu.SMEM),
        out_shape=jax.ShapeDtypeStruct((2,), jnp.float32),
    )


def kernel(input, target):
    n = input.shape[1]
    tgt_pl = jnp.transpose(target, (2, 0, 1))
    inp_pl = input.reshape(n // BLK, BLK, 2).transpose(0, 2, 1)
    part = _make_tc_partials(n)(inp_pl, tgt_pl, tgt_pl, tgt_pl)
    s = part[0]
    c = part[1]
    return jnp.where(c > 0, s / jnp.maximum(c, 1.0), jnp.float32(0.0))
